# Initial kernel scaffold; baseline (speedup 1.0000x reference)
import jax
import jax.numpy as jnp
from jax.experimental import pallas as pl


def kernel(x, edge_index, edge_attr, W1, b1, W2, b2, Wc, bc, Wmu, bmu, Wsig, bsig, high, deterministic):
    raise NotImplementedError("write your pallas kernel here")


# speedup vs baseline: 4.7128x; 4.7128x over previous
import functools

import jax
import jax.numpy as jnp
from jax import lax
from jax.experimental import pallas as pl
from jax.experimental.pallas import tpu as pltpu
from jax.experimental.pallas import tpu_sc as plsc

NC, NS = 2, 16
NW = NC * NS
_F32 = jnp.float32
_HI = lax.Precision.HIGHEST


def _gather_body(x_hbm, ii_hbm, jj_hbm, xi_hbm, xj_hbm,
                 iv, jv, ri, rj, s1, s2, *, epw, chunk):
    c = lax.axis_index("c")
    s = lax.axis_index("s")
    wid = s * NC + c

    def body(k, carry):
        base = wid * epw + k * chunk
        a = pltpu.async_copy(ii_hbm.at[pl.ds(base, chunk)], iv, s1)
        b = pltpu.async_copy(jj_hbm.at[pl.ds(base, chunk)], jv, s2)
        a.wait()
        b.wait()
        g1 = pltpu.async_copy(x_hbm.at[iv], ri, s1)
        g2 = pltpu.async_copy(x_hbm.at[jv], rj, s2)
        g1.wait()
        g2.wait()
        w1 = pltpu.async_copy(ri, xi_hbm.at[pl.ds(base, chunk)], s1)
        w2 = pltpu.async_copy(rj, xj_hbm.at[pl.ds(base, chunk)], s2)
        w1.wait()
        w2.wait()
        return carry

    lax.fori_loop(0, epw // chunk, body, 0)


def _scatter_body(z_hbm, msg_hbm, ii_hbm, out_hbm, iv, mv, acc,
                  *, n, hid, epw, chunk):
    c = lax.axis_index("c")
    s = lax.axis_index("s")
    wid = s * NC + c
    stripe = n // NS

    pltpu.sync_copy(z_hbm.at[pl.ds(s * stripe, stripe)],
                    acc.at[pl.ds(s * stripe, stripe)])
    plsc.subcore_barrier()

    def body(k, carry):
        base = wid * epw + k * chunk
        pltpu.sync_copy(ii_hbm.at[pl.ds(base, chunk)], iv)
        pltpu.sync_copy(msg_hbm.at[pl.ds(base, chunk)], mv)
        pltpu.sync_copy(mv, acc.at[iv], add=True)
        return carry

    lax.fori_loop(0, epw // chunk, body, 0)
    plsc.subcore_barrier()
    pltpu.sync_copy(acc.at[pl.ds(s * stripe, stripe)],
                    out_hbm.at[pl.ds(c * n + s * stripe, stripe)])


def _mlp_body(xi, xj, ea, w1a, w1b, w1c, b1, w2t, b2, out):
    h = (jnp.dot(xi[...], w1a[...], preferred_element_type=_F32)
         + jnp.dot(xj[...], w1b[...], preferred_element_type=_F32)
         + jnp.dot(ea[...], w1c[...], preferred_element_type=_F32)
         + b1[...])
    h = jnp.maximum(h, 0.0)
    out[...] = (jnp.dot(h, w2t[...], preferred_element_type=_F32)
                + b2[...])


def _softplus(z):
    return jnp.maximum(z, 0.0) + jnp.log(1.0 + jnp.exp(-jnp.abs(z)))


def _node_body(xr, pr, wbx, wbh, wmx, wmh, wsx, wsh,
               bc, bmu, bsig, highr, inv_out, ord_out):
    xpp = pr[0] + pr[1]
    xv = xr[...]
    zc = (jnp.dot(xv, wbx[...], preferred_element_type=_F32, precision=_HI)
          + jnp.dot(xpp, wbh[...], preferred_element_type=_F32, precision=_HI)
          + bc[...] + 1e-10)
    conc = _softplus(zc)
    total = jnp.sum(conc)
    inv_out[...] = conc / (total + 1e-20)
    zmu = (jnp.dot(xv, wmx[...], preferred_element_type=_F32, precision=_HI)
           + jnp.dot(xpp, wmh[...], preferred_element_type=_F32, precision=_HI)
           + bmu[...] + 1e-20)
    a = _softplus(zmu) + 1e-20
    zsg = (jnp.dot(xv, wsx[...], preferred_element_type=_F32, precision=_HI)
           + jnp.dot(xpp, wsh[...], preferred_element_type=_F32, precision=_HI)
           + bsig[...] + 1e-20)
    b = _softplus(zsg) + 1e-20
    ord_out[...] = a / (a + b) * highr[...]


def kernel(x, edge_index, edge_attr, W1, b1, W2, b2, Wc, bc, Wmu, bmu,
           Wsig, bsig, high, deterministic):
    N, NODE = x.shape
    E = edge_index.shape[1]
    EA = edge_attr.shape[1]
    HID = W2.shape[0]
    NF = high.shape[0]
    NN = 100
    G = N // NN

    ii = edge_index[0]
    jj = edge_index[1]

    epw = E // NW
    chunk = 2000
    mesh = plsc.VectorSubcoreMesh(core_axis_name="c", subcore_axis_name="s")
    sc_params = pltpu.CompilerParams(use_tc_tiling_on_sc=False)

    gather = pl.kernel(
        functools.partial(_gather_body, epw=epw, chunk=chunk),
        out_type=[jax.ShapeDtypeStruct((E, NODE), _F32),
                  jax.ShapeDtypeStruct((E, NODE), _F32)],
        mesh=mesh,
        scratch_types=[pltpu.VMEM((chunk,), jnp.int32),
                       pltpu.VMEM((chunk,), jnp.int32),
                       pltpu.VMEM((chunk, NODE), _F32),
                       pltpu.VMEM((chunk, NODE), _F32),
                       pltpu.SemaphoreType.DMA,
                       pltpu.SemaphoreType.DMA],
        compiler_params=sc_params,
    )
    xi, xj = gather(x, ii, jj)

    P = 8
    eye8 = jnp.eye(P, dtype=_F32)
    w1a = jnp.kron(eye8, W1[:, :NODE].T)
    w1b = jnp.kron(eye8, W1[:, NODE:2 * NODE].T)
    w1c = jnp.kron(eye8, W1[:, 2 * NODE:].T)
    w2t = jnp.kron(eye8, W2.T)
    b1r = jnp.tile(b1, P).reshape(1, P * HID)
    b2r = jnp.tile(b2, P).reshape(1, P * HID)
    xi_p = xi.reshape(E // P, P * NODE)
    xj_p = xj.reshape(E // P, P * NODE)
    ea_p = edge_attr.reshape(E // P, P * EA)
    BR = 2000
    ER = E // P
    msg_p = pl.pallas_call(
        _mlp_body,
        grid=(ER // BR,),
        in_specs=[
            pl.BlockSpec((BR, P * NODE), lambda e: (e, 0)),
            pl.BlockSpec((BR, P * NODE), lambda e: (e, 0)),
            pl.BlockSpec((BR, P * EA), lambda e: (e, 0)),
            pl.BlockSpec((P * NODE, P * HID), lambda e: (0, 0)),
            pl.BlockSpec((P * NODE, P * HID), lambda e: (0, 0)),
            pl.BlockSpec((P * EA, P * HID), lambda e: (0, 0)),
            pl.BlockSpec((1, P * HID), lambda e: (0, 0)),
            pl.BlockSpec((P * HID, P * HID), lambda e: (0, 0)),
            pl.BlockSpec((1, P * HID), lambda e: (0, 0)),
        ],
        out_specs=pl.BlockSpec((BR, P * HID), lambda e: (e, 0)),
        out_shape=jax.ShapeDtypeStruct((ER, P * HID), _F32),
    )(xi_p, xj_p, ea_p, w1a, w1b, w1c, b1r, w2t, b2r)
    msg = msg_p.reshape(E, HID)

    schunk = 400
    zeros = jnp.zeros((N, HID), _F32)
    scatter = pl.kernel(
        functools.partial(_scatter_body, n=N, hid=HID, epw=epw, chunk=schunk),
        out_type=jax.ShapeDtypeStruct((NC * N, HID), _F32),
        mesh=mesh,
        scratch_types=[pltpu.VMEM((schunk,), jnp.int32),
                       pltpu.VMEM((schunk, HID), _F32),
                       pltpu.VMEM_SHARED((N, HID), _F32)],
        compiler_params=sc_params,
    )
    part = scatter(zeros, msg, ii)

    xr = x.reshape(G, NN * NODE)
    pr = part.reshape(NC, G, NN * HID)
    eye = jnp.eye(NN, dtype=_F32)
    sel = (jnp.arange(NN)[:, None] == (NN - NF + jnp.arange(NF))[None, :])
    sel = sel.astype(_F32)
    wcx, wch = Wc[0, :NODE], Wc[0, NODE:]
    wmx, wmh = Wmu[0, :NODE], Wmu[0, NODE:]
    wsx, wsh = Wsig[0, :NODE], Wsig[0, NODE:]
    Wbx = jnp.kron(eye, wcx.reshape(NODE, 1))
    Wbh = jnp.kron(eye, wch.reshape(HID, 1))
    Wmx = jnp.kron(sel, wmx.reshape(NODE, 1))
    Wmh = jnp.kron(sel, wmh.reshape(HID, 1))
    Wsx = jnp.kron(sel, wsx.reshape(NODE, 1))
    Wsh = jnp.kron(sel, wsh.reshape(HID, 1))
    bcr = bc.reshape(1, 1)
    bmur = bmu.reshape(1, 1)
    bsigr = bsig.reshape(1, 1)
    highr = high.reshape(1, NF)

    inv, ordv = pl.pallas_call(
        _node_body,
        out_shape=[jax.ShapeDtypeStruct((G, NN), _F32),
                   jax.ShapeDtypeStruct((G, NF), _F32)],
        compiler_params=pltpu.CompilerParams(vmem_limit_bytes=63 << 20),
    )(xr, pr, Wbx, Wbh, Wmx, Wmh, Wsx, Wsh, bcr, bmur, bsigr, highr)

    return (inv, ordv)
